# baseline (speedup 1.0000x reference)
import functools

import jax
import jax.numpy as jnp
import numpy as np
from jax.experimental import pallas as pl
from jax.experimental.pallas import tpu as pltpu

N_NODES = 10000
_hist = np.array([0, 50, 150, 400, 800, 1500, 2200, 2000, 1500, 900, 400, 100], dtype=np.float32)
AVG_LOG_DEG = float((_hist * np.log(np.arange(_hist.shape[0]) + 1.0)).sum() / _hist.sum())

_BM = 512
_BN = 1280


def _pairwise_body(y_row, y_col, sq_row, sq_col, out_ref):
    a, b = 0.583, 1.334
    dot = jax.lax.dot_general(
        y_row[...], y_col[...], (((1,), (1,)), ((), ())),
        preferred_element_type=jnp.float32)
    d2 = jnp.maximum(sq_row[...] + sq_col[...].reshape(1, -1) - 2.0 * dot, 0.0)
    p = jnp.where(d2 > 0.0, jnp.exp(b * jnp.log(jnp.maximum(d2, 1e-30))), 0.0)
    out_ref[...] = 1.0 / (1.0 + a * p)


def _pairwise(y):
    n = y.shape[0]
    sq = jnp.sum(y * y, axis=1, keepdims=True)
    grid = (pl.cdiv(n, _BM), pl.cdiv(n, _BN))
    return pl.pallas_call(
        _pairwise_body,
        grid=grid,
        in_specs=[
            pl.BlockSpec((_BM, y.shape[1]), lambda i, j: (i, 0)),
            pl.BlockSpec((_BN, y.shape[1]), lambda i, j: (j, 0)),
            pl.BlockSpec((_BM, 1), lambda i, j: (i, 0)),
            pl.BlockSpec((_BN, 1), lambda i, j: (j, 0)),
        ],
        out_specs=pl.BlockSpec((_BM, _BN), lambda i, j: (i, j)),
        out_shape=jax.ShapeDtypeStruct((n, n), jnp.float32),
    )(y, y, sq, sq)


def kernel(x, edge_index, edge_attr, batch, params):
    N = x.shape[0]
    src = edge_index[0]
    dst = edge_index[1]
    h = x @ params['W_node'] + params['b_node']
    deg = jnp.zeros((N,), jnp.float32).at[dst].add(1.0)
    deg_c = jnp.maximum(deg, 1.0)[:, None]
    logdeg = jnp.log(deg + 1.0)
    amp = (logdeg / AVG_LOG_DEG)[:, None]
    att = (AVG_LOG_DEG / jnp.where(logdeg > 0.0, logdeg, 1.0))[:, None]
    for i in range(4):
        msgs = h[src]
        s1 = jax.ops.segment_sum(msgs, dst, num_segments=N)
        mean = s1 / deg_c
        s2 = jax.ops.segment_sum(msgs * msgs, dst, num_segments=N)
        mean2 = s2 / deg_c
        std = jnp.sqrt(jnp.maximum(mean2 - mean * mean, 0.0) + 1e-5)
        mn = jax.ops.segment_min(msgs, dst, num_segments=N)
        mx = jax.ops.segment_max(msgs, dst, num_segments=N)
        mn = jnp.where(jnp.isfinite(mn), mn, 0.0)
        mx = jnp.where(jnp.isfinite(mx), mx, 0.0)
        agg = jnp.concatenate([mean, mn, mx, std], axis=-1)
        out = jnp.concatenate([agg, agg * amp, agg * att], axis=-1)
        out = out @ params['W_post'][i] + params['b_post'][i]
        mu = out.mean(axis=0)
        var = out.var(axis=0)
        out = (out - mu) / jnp.sqrt(var + 1e-5) * params['gamma'][i] + params['beta'][i]
        h = jax.nn.relu(out) + h
    y = jax.nn.relu(h @ params['W1'] + params['b1'])
    y = jax.nn.relu(y @ params['W2'] + params['b2'])
    y = y @ params['W3'] + params['b3']
    return _pairwise(y)


# speedup vs baseline: 4.6765x; 4.6765x over previous
import functools

import jax
import jax.numpy as jnp
import numpy as np
from jax import lax
from jax.experimental import pallas as pl
from jax.experimental.pallas import tpu as pltpu
from jax.experimental.pallas import tpu_sc as plsc

N_NODES = 10000
N_EDGES = 640000
_hist = np.array([0, 50, 150, 400, 800, 1500, 2200, 2000, 1500, 900, 400, 100], dtype=np.float32)
AVG_LOG_DEG = float((_hist * np.log(np.arange(_hist.shape[0]) + 1.0)).sum() / _hist.sum())

_NC = 2
_NS = 16
_NW = _NC * _NS
_NPT = 313
_PAD = _NPT
_ACCR = _NPT + 1

_C = 2000
_FL = 4096
_S = 8192
_CAP = 160 * 4096
_K = 128

_mesh = plsc.VectorSubcoreMesh(core_axis_name="c", subcore_axis_name="s")


def _bucket_body(src_hbm, dst_hbm, bsrc_hbm, bldst_hbm, cnt_hbm, deg_hbm,
                 srcc, dstc, stsrc, stldst, dega, cntv, sem):
    wid = lax.axis_index("s") * _NC + lax.axis_index("c")
    lo = wid * _NPT
    zeros16 = jnp.zeros((16,), jnp.float32)

    def _initdeg(r, carry):
        dega[r, pl.ds(0, 16)] = zeros16
        return carry
    lax.fori_loop(0, _ACCR, _initdeg, 0)

    def _chunk(c, carry):
        w, goff = carry
        pltpu.sync_copy(src_hbm.at[pl.ds(pl.multiple_of(c * _C, 8), _C)], srcc)
        pltpu.sync_copy(dst_hbm.at[pl.ds(pl.multiple_of(c * _C, 8), _C)], dstc)

        def _vec(v, w):
            dv = dstc[pl.ds(v * 16, 16)]
            sv = srcc[pl.ds(v * 16, 16)]
            ldv = dv - lo
            m = (ldv >= 0) & (ldv < _NPT)
            plsc.store_compressed(stldst.at[pl.ds(w, 16)], ldv, mask=m)
            plsc.store_compressed(stsrc.at[pl.ds(w, 16)], sv, mask=m)
            return w + jnp.sum(m.astype(jnp.int32))
        w = lax.fori_loop(0, _C // 16, _vec, w)

        def _flush(args):
            w, goff = args
            pltpu.sync_copy(stsrc.at[pl.ds(0, _FL)], bsrc_hbm.at[wid, pl.ds(pl.multiple_of(goff, 8), _FL)])
            pltpu.sync_copy(stldst.at[pl.ds(0, _FL)], bldst_hbm.at[wid, pl.ds(pl.multiple_of(goff, 8), _FL)])

            def _mv(i, carry):
                stsrc[pl.ds(i * 16, 16)] = stsrc[pl.ds(_FL + i * 16, 16)]
                stldst[pl.ds(i * 16, 16)] = stldst[pl.ds(_FL + i * 16, 16)]
                return carry
            lax.fori_loop(0, (_C + 16) // 16 + 1, _mv, 0)
            return w - _FL, goff + _FL

        w, goff = lax.cond(w >= _FL, _flush, lambda a: a, (w, goff))
        return w, goff

    w, goff = lax.fori_loop(0, N_EDGES // _C, _chunk, (0, 0))

    iota16 = lax.iota(jnp.int32, 16)
    padl16 = jnp.full((16,), _PAD, jnp.int32)
    zi16 = jnp.zeros((16,), jnp.int32)

    def _pad(i, w):
        base = w + i * 16
        plsc.store_scatter(stldst, [base + iota16], padl16)
        plsc.store_scatter(stsrc, [base + iota16], zi16)
        return w
    lax.fori_loop(0, 10, _pad, w)

    pltpu.sync_copy(stsrc.at[pl.ds(0, _FL)], bsrc_hbm.at[wid, pl.ds(pl.multiple_of(goff, 8), _FL)])
    pltpu.sync_copy(stldst.at[pl.ds(0, _FL)], bldst_hbm.at[wid, pl.ds(pl.multiple_of(goff, 8), _FL)])
    pltpu.sync_copy(stsrc.at[pl.ds(_FL, _FL)], bsrc_hbm.at[wid, pl.ds(pl.multiple_of(goff + _FL, 8), _FL)])
    pltpu.sync_copy(stldst.at[pl.ds(_FL, _FL)], bldst_hbm.at[wid, pl.ds(pl.multiple_of(goff + _FL, 8), _FL)])

    total = goff + w
    cntv[...] = jnp.full((16,), total, jnp.int32)
    pltpu.sync_copy(cntv.at[pl.ds(0, 8)], cnt_hbm.at[wid])

    iota16i = lax.iota(jnp.int32, 16)
    e0 = (iota16i == 0).astype(jnp.float32)
    nchunks = (total + _K - 1) // _K

    def _dchunk(ci, carry):
        off = ci * _K
        pltpu.sync_copy(bldst_hbm.at[wid, pl.ds(pl.multiple_of(off, 8), _K)],
                        srcc.at[pl.ds(0, _K)])

        def _dedge(e, carry):
            d = srcc[pl.ds(e, 16)][0]
            dega[d, pl.ds(0, 16)] = dega[d, pl.ds(0, 16)] + e0
            return carry
        lax.fori_loop(0, _K, _dedge, 0)
        return carry
    lax.fori_loop(0, nchunks, _dchunk, 0)
    pltpu.sync_copy(dega, deg_hbm.at[wid])


_bucket = functools.partial(
    pl.kernel,
    out_type=[
        jax.ShapeDtypeStruct((_NW, _CAP), jnp.int32),
        jax.ShapeDtypeStruct((_NW, _CAP), jnp.int32),
        jax.ShapeDtypeStruct((_NW, 8), jnp.int32),
        jax.ShapeDtypeStruct((_NW, _ACCR, 16), jnp.float32),
    ],
    mesh=_mesh,
    compiler_params=pltpu.CompilerParams(use_tc_tiling_on_sc=False, needs_layout_passes=False),
    scratch_types=[
        pltpu.VMEM((_C,), jnp.int32),
        pltpu.VMEM((_C,), jnp.int32),
        pltpu.VMEM((_S,), jnp.int32),
        pltpu.VMEM((_S,), jnp.int32),
        pltpu.VMEM((_ACCR, 16), jnp.float32),
        pltpu.VMEM((16,), jnp.int32),
        pltpu.SemaphoreType.DMA,
    ],
)(_bucket_body)


def _seg_body(h_hbm, bsrc_hbm, bldst_hbm, cnt_hbm, out_hbm,
              acc_s, acc_q, acc_mn, acc_mx, msgs, sidx, lidx, cntv, sem):
    wid = lax.axis_index("s") * _NC + lax.axis_index("c")
    zeros16 = jnp.zeros((16,), jnp.float32)
    big16 = jnp.full((16,), 3.0e38, jnp.float32)

    pltpu.sync_copy(cnt_hbm.at[wid], cntv.at[pl.ds(0, 8)])
    total = cntv[pl.ds(0, 16)][0]
    nchunks = (total + _K - 1) // _K

    def _init(r, carry):
        for f in range(5):
            sl = pl.ds(f * 16, 16)
            acc_s[r, sl] = zeros16
            acc_q[r, sl] = zeros16
            acc_mn[r, sl] = big16
            acc_mx[r, sl] = -big16
        return carry
    lax.fori_loop(0, _ACCR, _init, 0)

    def _chunk(ci, carry):
        off = ci * _K
        pltpu.sync_copy(bsrc_hbm.at[wid, pl.ds(pl.multiple_of(off, 8), _K)], sidx)
        pltpu.sync_copy(bldst_hbm.at[wid, pl.ds(pl.multiple_of(off, 8), _K)], lidx.at[pl.ds(0, _K)])
        pltpu.async_copy(h_hbm.at[sidx], msgs, sem).wait()

        def _edge(e, carry):
            d = lidx[pl.ds(e, 16)][0]
            for f in range(5):
                sl = pl.ds(f * 16, 16)
                m = msgs[e, sl]
                acc_s[d, sl] = acc_s[d, sl] + m
                acc_q[d, sl] = acc_q[d, sl] + m * m
                acc_mn[d, sl] = jnp.minimum(acc_mn[d, sl], m)
                acc_mx[d, sl] = jnp.maximum(acc_mx[d, sl], m)
            return carry
        lax.fori_loop(0, _K, _edge, 0)
        return carry

    lax.fori_loop(0, nchunks, _chunk, 0)

    pltpu.sync_copy(acc_s, out_hbm.at[0, wid])
    pltpu.sync_copy(acc_q, out_hbm.at[1, wid])
    pltpu.sync_copy(acc_mn, out_hbm.at[2, wid])
    pltpu.sync_copy(acc_mx, out_hbm.at[3, wid])


_seg = functools.partial(
    pl.kernel,
    out_type=jax.ShapeDtypeStruct((4, _NW, _ACCR, 80), jnp.float32),
    mesh=_mesh,
    compiler_params=pltpu.CompilerParams(use_tc_tiling_on_sc=False, needs_layout_passes=False),
    scratch_types=[
        pltpu.VMEM((_ACCR, 80), jnp.float32),
        pltpu.VMEM((_ACCR, 80), jnp.float32),
        pltpu.VMEM((_ACCR, 80), jnp.float32),
        pltpu.VMEM((_ACCR, 80), jnp.float32),
        pltpu.VMEM((_K, 80), jnp.float32),
        pltpu.VMEM((_K,), jnp.int32),
        pltpu.VMEM((_K + 16,), jnp.int32),
        pltpu.VMEM((16,), jnp.int32),
        pltpu.SemaphoreType.DMA,
    ],
)(_seg_body)


_BM = 512
_BN = 1280


def _pairwise_body(y_row, y_col, sq_row, sq_col, out_ref):
    a, b = 0.583, 1.334
    dot = lax.dot_general(
        y_row[...], y_col[...], (((1,), (1,)), ((), ())),
        preferred_element_type=jnp.float32)
    d2 = jnp.maximum(sq_row[...] + sq_col[...].reshape(1, -1) - 2.0 * dot, 0.0)
    p = jnp.where(d2 > 0.0, jnp.exp(b * jnp.log(jnp.maximum(d2, 1e-30))), 0.0)
    out_ref[...] = 1.0 / (1.0 + a * p)


def _pairwise(y):
    n = y.shape[0]
    sq = jnp.sum(y * y, axis=1, keepdims=True)
    grid = (pl.cdiv(n, _BM), pl.cdiv(n, _BN))
    return pl.pallas_call(
        _pairwise_body,
        grid=grid,
        in_specs=[
            pl.BlockSpec((_BM, y.shape[1]), lambda i, j: (i, 0)),
            pl.BlockSpec((_BN, y.shape[1]), lambda i, j: (j, 0)),
            pl.BlockSpec((_BM, 1), lambda i, j: (i, 0)),
            pl.BlockSpec((_BN, 1), lambda i, j: (j, 0)),
        ],
        out_specs=pl.BlockSpec((_BM, _BN), lambda i, j: (i, j)),
        out_shape=jax.ShapeDtypeStruct((n, n), jnp.float32),
    )(y, y, sq, sq)


def kernel(x, edge_index, edge_attr, batch, params):
    N = x.shape[0]
    src = edge_index[0]
    dst = edge_index[1]
    h = x @ params['W_node'] + params['b_node']

    bsrc, bldst, cnts, deg_slab = _bucket(src, dst)
    deg = deg_slab[:, :_NPT, 0].reshape(-1)[:N]
    deg_c = jnp.maximum(deg, 1.0)[:, None]
    has_edge = (deg > 0.0)[:, None]
    logdeg = jnp.log(deg + 1.0)
    amp = (logdeg / AVG_LOG_DEG)[:, None]
    att = (AVG_LOG_DEG / jnp.where(logdeg > 0.0, logdeg, 1.0))[:, None]

    for i in range(4):
        stats = _seg(h, bsrc, bldst, cnts)
        st = stats[:, :, :_NPT, :].reshape(4, _NW * _NPT, 80)[:, :N, :]
        s1, s2, mn, mx = st[0], st[1], st[2], st[3]
        mean = s1 / deg_c
        mean2 = s2 / deg_c
        std = jnp.sqrt(jnp.maximum(mean2 - mean * mean, 0.0) + 1e-5)
        mn = jnp.where(has_edge, mn, 0.0)
        mx = jnp.where(has_edge, mx, 0.0)
        agg = jnp.concatenate([mean, mn, mx, std], axis=-1)
        out = jnp.concatenate([agg, agg * amp, agg * att], axis=-1)
        out = out @ params['W_post'][i] + params['b_post'][i]
        mu = out.mean(axis=0)
        var = out.var(axis=0)
        out = (out - mu) / jnp.sqrt(var + 1e-5) * params['gamma'][i] + params['beta'][i]
        h = jax.nn.relu(out) + h
    y = jax.nn.relu(h @ params['W1'] + params['b1'])
    y = jax.nn.relu(y @ params['W2'] + params['b2'])
    y = y @ params['W3'] + params['b3']
    return _pairwise(y)
